# baseline (speedup 1.0000x reference)
import functools

import jax
import jax.numpy as jnp
from jax import lax
from jax.experimental import pallas as pl
from jax.experimental.pallas import tpu as pltpu

_NUM_SEGMENTS = 256


def _pool_body(x_ref, ind_ref, wa_ref, bf_ref, wt_ref, bt_ref,
               out_ref, m_ref, p_ref, *, block_rows):
    B = block_rows
    S = _NUM_SEGMENTS
    D = x_ref.shape[1]
    i = pl.program_id(0)
    nb = pl.num_programs(0)

    @pl.when(i == 0)
    def _init():
        m_ref[0] = 0.0
        p_ref[...] = jnp.zeros((S, D + 1), jnp.float32)

    xb = x_ref[...]
    ind_row = ind_ref[...].reshape(1, B)

    raw = jnp.dot(xb, wa_ref[...], preferred_element_type=jnp.float32)
    gate = raw[:, D:D + 1]
    feat = raw[:, :D] + bf_ref[...]
    feat = jnp.maximum(feat, 0.01 * feat)

    m_old = m_ref[0]
    w_col = jnp.exp(gate - m_old)
    feat_aug = jnp.concatenate([feat * w_col, w_col], axis=1)

    seg_iota = lax.broadcasted_iota(jnp.int32, (S, 1), 0)
    oh = jnp.where(seg_iota == ind_row, jnp.float32(1), jnp.float32(0))
    p_loc = jnp.dot(oh.astype(jnp.bfloat16), feat_aug.astype(jnp.bfloat16),
                    preferred_element_type=jnp.float32)

    m_new = jnp.maximum(m_old, jnp.max(gate))
    s = jnp.exp(m_old - m_new)
    p_ref[...] = (p_ref[...] + p_loc) * s
    m_ref[0] = m_new

    @pl.when(i == nb - 1)
    def _final():
        acc = p_ref[...]
        pooled = acc[:, :D] / (acc[:, D:D + 1] + 1e-16)
        o = jnp.dot(pooled, wt_ref[...],
                    preferred_element_type=jnp.float32) + bt_ref[...]
        out_ref[...] = jnp.maximum(o, 0.01 * o)


def kernel(x, batch_ind, W_mask, b_mask, W_feat, b_feat, W_trans, b_trans):
    del b_mask
    N, D = x.shape
    S = _NUM_SEGMENTS
    B = 2000 if N % 2000 == 0 else 2048
    nb = -(-N // B)

    ind = batch_ind.astype(jnp.int32)
    if nb * B != N:
        x = jnp.pad(x, ((0, nb * B - N), (0, 0)))
        ind = jnp.pad(ind, (0, nb * B - N), constant_values=-1)
    ind3 = ind.reshape(nb, 1, B)
    W_aug = jnp.concatenate([W_feat, W_mask], axis=1)

    body = functools.partial(_pool_body, block_rows=B)
    out = pl.pallas_call(
        body,
        grid=(nb,),
        in_specs=[
            pl.BlockSpec((B, D), lambda i: (i, 0)),
            pl.BlockSpec((1, 1, B), lambda i: (i, 0, 0)),
            pl.BlockSpec((D, D + 1), lambda i: (0, 0)),
            pl.BlockSpec((1, D), lambda i: (0, 0)),
            pl.BlockSpec((D, D), lambda i: (0, 0)),
            pl.BlockSpec((1, D), lambda i: (0, 0)),
        ],
        out_specs=pl.BlockSpec((S, D), lambda i: (0, 0)),
        out_shape=jax.ShapeDtypeStruct((S, D), jnp.float32),
        scratch_shapes=[
            pltpu.SMEM((1,), jnp.float32),
            pltpu.VMEM((S, D + 1), jnp.float32),
        ],
        compiler_params=pltpu.CompilerParams(
            dimension_semantics=("arbitrary",)),
    )(x, ind3, W_aug, b_feat.reshape(1, D), W_trans, b_trans.reshape(1, D))
    return out


# speedup vs baseline: 1.4268x; 1.4268x over previous
import functools

import jax
import jax.numpy as jnp
from jax import lax
from jax.experimental import pallas as pl
from jax.experimental.pallas import tpu as pltpu

_NUM_SEGMENTS = 256


def _pool_body(x_ref, ind_ref, wa_ref, bf_ref, wt_ref, bt_ref,
               out_ref, m_ref, p_ref, *, block_rows):
    B = block_rows
    S = _NUM_SEGMENTS
    D = x_ref.shape[1]
    i = pl.program_id(0)
    nb = pl.num_programs(0)

    @pl.when(i == 0)
    def _init():
        m_ref[0] = 0.0
        p_ref[...] = jnp.zeros((S, D + 1), jnp.float32)

    xb = x_ref[...]
    ind_row = ind_ref[...].reshape(1, B)

    raw = jnp.dot(xb, wa_ref[...], preferred_element_type=jnp.float32)
    gate = raw[:, D:D + 1]
    feat = raw[:, :D] + bf_ref[...]
    feat = jnp.maximum(feat, 0.01 * feat)

    m_old = m_ref[0]
    w_col = jnp.exp(gate - m_old)
    feat_aug = jnp.concatenate([feat * w_col, w_col], axis=1)

    seg_iota = lax.broadcasted_iota(jnp.int32, (S, 1), 0)
    oh = jnp.where(seg_iota == ind_row, jnp.float32(1), jnp.float32(0))
    p_loc = jnp.dot(oh.astype(jnp.bfloat16), feat_aug.astype(jnp.bfloat16),
                    preferred_element_type=jnp.float32)

    m_new = jnp.maximum(m_old, jnp.max(gate))
    s = jnp.exp(m_old - m_new)
    p_ref[...] = (p_ref[...] + p_loc) * s
    m_ref[0] = m_new

    @pl.when(i == nb - 1)
    def _final():
        acc = p_ref[...]
        pooled = acc[:, :D] / (acc[:, D:D + 1] + 1e-16)
        o = jnp.dot(pooled, wt_ref[...],
                    preferred_element_type=jnp.float32) + bt_ref[...]
        out_ref[...] = jnp.maximum(o, 0.01 * o)


def kernel(x, batch_ind, W_mask, b_mask, W_feat, b_feat, W_trans, b_trans):
    del b_mask
    N, D = x.shape
    S = _NUM_SEGMENTS
    B = 5000 if N % 5000 == 0 else 2048
    nb = -(-N // B)

    ind = batch_ind.astype(jnp.int32)
    if nb * B != N:
        x = jnp.pad(x, ((0, nb * B - N), (0, 0)))
        ind = jnp.pad(ind, (0, nb * B - N), constant_values=-1)
    ind3 = ind.reshape(nb, 1, B)
    W_aug = jnp.concatenate([W_feat, W_mask], axis=1)

    body = functools.partial(_pool_body, block_rows=B)
    out = pl.pallas_call(
        body,
        grid=(nb,),
        in_specs=[
            pl.BlockSpec((B, D), lambda i: (i, 0)),
            pl.BlockSpec((1, 1, B), lambda i: (i, 0, 0)),
            pl.BlockSpec((D, D + 1), lambda i: (0, 0)),
            pl.BlockSpec((1, D), lambda i: (0, 0)),
            pl.BlockSpec((D, D), lambda i: (0, 0)),
            pl.BlockSpec((1, D), lambda i: (0, 0)),
        ],
        out_specs=pl.BlockSpec((S, D), lambda i: (0, 0)),
        out_shape=jax.ShapeDtypeStruct((S, D), jnp.float32),
        scratch_shapes=[
            pltpu.SMEM((1,), jnp.float32),
            pltpu.VMEM((S, D + 1), jnp.float32),
        ],
        compiler_params=pltpu.CompilerParams(
            dimension_semantics=("arbitrary",)),
    )(x, ind3, W_aug, b_feat.reshape(1, D), W_trans, b_trans.reshape(1, D))
    return out


# speedup vs baseline: 1.5746x; 1.1036x over previous
import functools

import jax
import jax.numpy as jnp
from jax import lax
from jax.experimental import pallas as pl
from jax.experimental.pallas import tpu as pltpu

_NUM_SEGMENTS = 256


def _pool_body(x_ref, ind_ref, wa_ref, bf_ref, wt_ref, bt_ref,
               out_ref, m_ref, p_ref, *, block_rows):
    B = block_rows
    S = _NUM_SEGMENTS
    D = x_ref.shape[1]
    i = pl.program_id(0)
    nb = pl.num_programs(0)

    @pl.when(i == 0)
    def _init():
        m_ref[0] = 0.0
        p_ref[...] = jnp.zeros((S, D + 1), jnp.float32)

    xb = x_ref[...]
    ind_row = ind_ref[...].reshape(1, B)

    raw = jnp.dot(xb, wa_ref[...], preferred_element_type=jnp.float32)
    gate = raw[:, D:D + 1]
    feat = raw[:, :D] + bf_ref[...]
    feat = jnp.maximum(feat, 0.01 * feat)

    m_old = m_ref[0]
    w_col = jnp.exp(gate - m_old)
    feat_aug = jnp.concatenate([feat * w_col, w_col], axis=1)

    seg_iota = lax.broadcasted_iota(jnp.int32, (S, 1), 0)
    oh = jnp.where(seg_iota == ind_row, jnp.float32(1), jnp.float32(0))
    p_loc = jnp.dot(oh.astype(jnp.bfloat16), feat_aug.astype(jnp.bfloat16),
                    preferred_element_type=jnp.float32)

    m_new = jnp.maximum(m_old, jnp.max(gate))
    s = jnp.exp(m_old - m_new)
    p_ref[...] = (p_ref[...] + p_loc) * s
    m_ref[0] = m_new

    @pl.when(i == nb - 1)
    def _final():
        acc = p_ref[...]
        pooled = acc[:, :D] / (acc[:, D:D + 1] + 1e-16)
        o = jnp.dot(pooled, wt_ref[...],
                    preferred_element_type=jnp.float32) + bt_ref[...]
        out_ref[...] = jnp.maximum(o, 0.01 * o)


def kernel(x, batch_ind, W_mask, b_mask, W_feat, b_feat, W_trans, b_trans):
    del b_mask
    N, D = x.shape
    S = _NUM_SEGMENTS
    B = 10000 if N % 10000 == 0 else 2048
    nb = -(-N // B)

    ind = batch_ind.astype(jnp.int32)
    if nb * B != N:
        x = jnp.pad(x, ((0, nb * B - N), (0, 0)))
        ind = jnp.pad(ind, (0, nb * B - N), constant_values=-1)
    ind3 = ind.reshape(nb, 1, B)
    W_aug = jnp.concatenate([W_feat, W_mask], axis=1)

    body = functools.partial(_pool_body, block_rows=B)
    out = pl.pallas_call(
        body,
        grid=(nb,),
        in_specs=[
            pl.BlockSpec((B, D), lambda i: (i, 0)),
            pl.BlockSpec((1, 1, B), lambda i: (i, 0, 0)),
            pl.BlockSpec((D, D + 1), lambda i: (0, 0)),
            pl.BlockSpec((1, D), lambda i: (0, 0)),
            pl.BlockSpec((D, D), lambda i: (0, 0)),
            pl.BlockSpec((1, D), lambda i: (0, 0)),
        ],
        out_specs=pl.BlockSpec((S, D), lambda i: (0, 0)),
        out_shape=jax.ShapeDtypeStruct((S, D), jnp.float32),
        scratch_shapes=[
            pltpu.SMEM((1,), jnp.float32),
            pltpu.VMEM((S, D + 1), jnp.float32),
        ],
        compiler_params=pltpu.CompilerParams(
            dimension_semantics=("arbitrary",)),
    )(x, ind3, W_aug, b_feat.reshape(1, D), W_trans, b_trans.reshape(1, D))
    return out


# speedup vs baseline: 1.6065x; 1.0202x over previous
import functools

import jax
import jax.numpy as jnp
from jax import lax
from jax.experimental import pallas as pl
from jax.experimental.pallas import tpu as pltpu

_NUM_SEGMENTS = 256


def _pool_body(x_ref, ind_ref, wa_ref, bf_ref, wt_ref, bt_ref,
               out_ref, m_ref, p_ref, *, block_rows):
    B = block_rows
    S = _NUM_SEGMENTS
    D = x_ref.shape[1]
    i = pl.program_id(0)
    nb = pl.num_programs(0)

    @pl.when(i == 0)
    def _init():
        m_ref[0] = 0.0
        p_ref[...] = jnp.zeros((S, D + 1), jnp.float32)

    xb = x_ref[...]
    ind_row = ind_ref[...].reshape(1, B)

    raw = jnp.dot(xb, wa_ref[...], preferred_element_type=jnp.float32)
    gate = raw[:, D:D + 1]
    feat = raw[:, :D] + bf_ref[...]
    feat = jnp.maximum(feat, 0.01 * feat)

    m_old = m_ref[0]
    w_col = jnp.exp(gate - m_old)
    feat_aug = jnp.concatenate([feat * w_col, w_col], axis=1)

    seg_iota = lax.broadcasted_iota(jnp.int32, (S, 1), 0)
    oh = jnp.where(seg_iota == ind_row, jnp.float32(1), jnp.float32(0))
    p_loc = jnp.dot(oh.astype(jnp.bfloat16), feat_aug.astype(jnp.bfloat16),
                    preferred_element_type=jnp.float32)

    m_new = jnp.maximum(m_old, jnp.max(gate))
    s = jnp.exp(m_old - m_new)
    p_ref[...] = (p_ref[...] + p_loc) * s
    m_ref[0] = m_new

    @pl.when(i == nb - 1)
    def _final():
        acc = p_ref[...]
        pooled = acc[:, :D] / (acc[:, D:D + 1] + 1e-16)
        o = jnp.dot(pooled, wt_ref[...],
                    preferred_element_type=jnp.float32) + bt_ref[...]
        out_ref[...] = jnp.maximum(o, 0.01 * o)


def kernel(x, batch_ind, W_mask, b_mask, W_feat, b_feat, W_trans, b_trans):
    del b_mask
    N, D = x.shape
    S = _NUM_SEGMENTS
    B = 20000 if N % 20000 == 0 else 2048
    nb = -(-N // B)

    ind = batch_ind.astype(jnp.int32)
    if nb * B != N:
        x = jnp.pad(x, ((0, nb * B - N), (0, 0)))
        ind = jnp.pad(ind, (0, nb * B - N), constant_values=-1)
    ind3 = ind.reshape(nb, 1, B)
    W_aug = jnp.concatenate([W_feat, W_mask], axis=1)

    body = functools.partial(_pool_body, block_rows=B)
    out = pl.pallas_call(
        body,
        grid=(nb,),
        in_specs=[
            pl.BlockSpec((B, D), lambda i: (i, 0)),
            pl.BlockSpec((1, 1, B), lambda i: (i, 0, 0)),
            pl.BlockSpec((D, D + 1), lambda i: (0, 0)),
            pl.BlockSpec((1, D), lambda i: (0, 0)),
            pl.BlockSpec((D, D), lambda i: (0, 0)),
            pl.BlockSpec((1, D), lambda i: (0, 0)),
        ],
        out_specs=pl.BlockSpec((S, D), lambda i: (0, 0)),
        out_shape=jax.ShapeDtypeStruct((S, D), jnp.float32),
        scratch_shapes=[
            pltpu.SMEM((1,), jnp.float32),
            pltpu.VMEM((S, D + 1), jnp.float32),
        ],
        compiler_params=pltpu.CompilerParams(
            dimension_semantics=("arbitrary",)),
    )(x, ind3, W_aug, b_feat.reshape(1, D), W_trans, b_trans.reshape(1, D))
    return out
